# baseline (speedup 1.0000x reference)
import jax
import jax.numpy as jnp
from jax.experimental import pallas as pl
from jax.experimental.pallas import tpu as pltpu

T_TILE = 2048


def _router_body(x_ref, w1_ref, b1_ref, w2_ref, b2_ref, wg_ref,
                 logits_ref, probs_ref, tki_ref, tkp_ref, feat_ref,
                 lg_ref):
    i = pl.program_id(0)
    n = pl.num_programs(0)

    @pl.when(i > 0)
    def _epilogue():
        logits = lg_ref[...]
        m = jnp.max(logits, axis=0, keepdims=True)
        e = jnp.exp(logits - m)
        s = jnp.sum(e, axis=0, keepdims=True)
        probs_ref[...] = e / s

        n_e, t = logits.shape
        iota = jax.lax.broadcasted_iota(jnp.int32, (n_e, t), 0)
        pow2 = (1 << iota).astype(jnp.float32)
        eq1 = (logits == m).astype(jnp.float32)
        bits1 = jnp.sum(eq1 * pow2, axis=0, keepdims=True).astype(jnp.int32)
        lsb1 = bits1 & (-bits1)
        i1 = (jax.lax.bitcast_convert_type(lsb1.astype(jnp.float32),
                                           jnp.int32) >> 23) - 127
        masked = jnp.where(iota == i1, -jnp.inf, logits)
        m2 = jnp.max(masked, axis=0, keepdims=True)
        eq2 = (masked == m2).astype(jnp.float32)
        bits2 = jnp.sum(eq2 * pow2, axis=0, keepdims=True).astype(jnp.int32)
        lsb2 = bits2 & (-bits2)
        i2 = (jax.lax.bitcast_convert_type(lsb2.astype(jnp.float32),
                                           jnp.int32) >> 23) - 127
        tki_ref[...] = jnp.concatenate([i1, i2], axis=0)
        p1 = 1.0 / s
        p2 = jnp.exp(m2 - m) / s
        denom = p1 + p2
        tkp_ref[...] = jnp.concatenate([p1 / denom, p2 / denom], axis=0)

    @pl.when(i < n - 1)
    def _matmuls():
        h = jnp.maximum(
            jnp.dot(x_ref[...], w1_ref[...],
                    preferred_element_type=jnp.float32) + b1_ref[...], 0.0)
        h = jnp.maximum(
            jnp.dot(h, w2_ref[...],
                    preferred_element_type=jnp.float32) + b2_ref[...], 0.0)
        feat_ref[...] = h
        logits = jax.lax.dot_general(
            wg_ref[...], h, (((0,), (1,)), ((), ())),
            preferred_element_type=jnp.float32)
        logits_ref[...] = logits
        lg_ref[...] = logits


@jax.jit
def kernel(x, W1, b1, W2, b2, Wg):
    ntok, hidden = x.shape
    rhid = W1.shape[1]
    n_e = Wg.shape[1]
    grid = ntok // T_TILE

    out_shapes = (
        jax.ShapeDtypeStruct((n_e, ntok), jnp.float32),
        jax.ShapeDtypeStruct((n_e, ntok), jnp.float32),
        jax.ShapeDtypeStruct((2, ntok), jnp.int32),
        jax.ShapeDtypeStruct((2, ntok), jnp.float32),
        jax.ShapeDtypeStruct((ntok, rhid), jnp.float32),
    )
    last = grid - 1
    cur = lambda i: jnp.minimum(i, last)
    prev = lambda i: jnp.maximum(i - 1, 0)

    lt, pt, kit, kpt, feat = pl.pallas_call(
        _router_body,
        grid=(grid + 1,),
        in_specs=[
            pl.BlockSpec((T_TILE, hidden), lambda i: (cur(i), 0)),
            pl.BlockSpec((hidden, rhid), lambda i: (0, 0)),
            pl.BlockSpec((1, rhid), lambda i: (0, 0)),
            pl.BlockSpec((rhid, rhid), lambda i: (0, 0)),
            pl.BlockSpec((1, rhid), lambda i: (0, 0)),
            pl.BlockSpec((rhid, n_e), lambda i: (0, 0)),
        ],
        out_specs=(
            pl.BlockSpec((n_e, T_TILE), lambda i: (0, cur(i))),
            pl.BlockSpec((n_e, T_TILE), lambda i: (0, prev(i))),
            pl.BlockSpec((2, T_TILE), lambda i: (0, prev(i))),
            pl.BlockSpec((2, T_TILE), lambda i: (0, prev(i))),
            pl.BlockSpec((T_TILE, rhid), lambda i: (cur(i), 0)),
        ),
        out_shape=out_shapes,
        scratch_shapes=[pltpu.VMEM((n_e, T_TILE), jnp.float32)],
        compiler_params=pltpu.CompilerParams(
            dimension_semantics=("arbitrary",)),
    )(x, W1, b1.reshape(1, -1), W2, b2.reshape(1, -1), Wg)
    return lt.T, pt.T, kit.T, kpt.T, feat


# speedup vs baseline: 1.0071x; 1.0071x over previous
import jax
import jax.numpy as jnp
from jax.experimental import pallas as pl
from jax.experimental.pallas import tpu as pltpu

T_TILE = 2048


HALVES = 2


def _router_body(x_ref, w1_ref, b1_ref, w2_ref, b2_ref,
                 wg_ref, logits_ref, probs_ref, tki_ref, tkp_ref, feat_ref):
    th = x_ref.shape[0] // HALVES
    for half in range(HALVES):
        rows = slice(half * th, (half + 1) * th)
        h = jnp.maximum(
            jnp.dot(x_ref[rows, :], w1_ref[...],
                    preferred_element_type=jnp.float32) + b1_ref[...], 0.0)
        h = jnp.maximum(
            jnp.dot(h, w2_ref[...],
                    preferred_element_type=jnp.float32) + b2_ref[...], 0.0)
        feat_ref[rows, :] = h
        logits = jax.lax.dot_general(
            wg_ref[...], h, (((0,), (1,)), ((), ())),
            preferred_element_type=jnp.float32)
        logits_ref[:, rows] = logits

        m = jnp.max(logits, axis=0, keepdims=True)
        e = jnp.exp(logits - m)
        s = jnp.sum(e, axis=0, keepdims=True)
        probs_ref[:, rows] = e / s

        n_e, t = logits.shape
        iota = jax.lax.broadcasted_iota(jnp.int32, (n_e, t), 0)
        pow2 = (1 << iota).astype(jnp.float32)
        eq1 = (logits == m).astype(jnp.float32)
        bits1 = jnp.sum(eq1 * pow2, axis=0, keepdims=True).astype(jnp.int32)
        lsb1 = bits1 & (-bits1)
        i1 = (jax.lax.bitcast_convert_type(lsb1.astype(jnp.float32),
                                           jnp.int32) >> 23) - 127
        masked = jnp.where(iota == i1, -jnp.inf, logits)
        m2 = jnp.max(masked, axis=0, keepdims=True)
        eq2 = (masked == m2).astype(jnp.float32)
        bits2 = jnp.sum(eq2 * pow2, axis=0, keepdims=True).astype(jnp.int32)
        lsb2 = bits2 & (-bits2)
        i2 = (jax.lax.bitcast_convert_type(lsb2.astype(jnp.float32),
                                           jnp.int32) >> 23) - 127
        tki_ref[:, rows] = jnp.concatenate([i1, i2], axis=0)
        p1 = 1.0 / s
        p2 = jnp.exp(m2 - m) / s
        denom = p1 + p2
        tkp_ref[:, rows] = jnp.concatenate([p1 / denom, p2 / denom], axis=0)


@jax.jit
def kernel(x, W1, b1, W2, b2, Wg):
    ntok, hidden = x.shape
    rhid = W1.shape[1]
    n_e = Wg.shape[1]
    grid = ntok // T_TILE

    out_shapes = (
        jax.ShapeDtypeStruct((n_e, ntok), jnp.float32),
        jax.ShapeDtypeStruct((n_e, ntok), jnp.float32),
        jax.ShapeDtypeStruct((2, ntok), jnp.int32),
        jax.ShapeDtypeStruct((2, ntok), jnp.float32),
        jax.ShapeDtypeStruct((ntok, rhid), jnp.float32),
    )
    tok_spec = lambda w: pl.BlockSpec((T_TILE, w), lambda i: (i, 0))
    tr_spec = lambda rows: pl.BlockSpec((rows, T_TILE), lambda i: (0, i))
    fixed_spec = lambda a, b: pl.BlockSpec((a, b), lambda i: (0, 0))

    lt, pt, kit, kpt, feat = pl.pallas_call(
        _router_body,
        grid=(grid,),
        in_specs=[
            tok_spec(hidden),
            fixed_spec(hidden, rhid),
            fixed_spec(1, rhid),
            fixed_spec(rhid, rhid),
            fixed_spec(1, rhid),
            fixed_spec(rhid, n_e),
        ],
        out_specs=(
            tr_spec(n_e),
            tr_spec(n_e),
            tr_spec(2),
            tr_spec(2),
            tok_spec(rhid),
        ),
        out_shape=out_shapes,
        compiler_params=pltpu.CompilerParams(
            dimension_semantics=("parallel",)),
    )(x, W1, b1.reshape(1, -1), W2, b2.reshape(1, -1), Wg)
    return lt.T, pt.T, kit.T, kpt.T, feat
